# baseline (speedup 1.0000x reference)
import functools

import jax
import jax.numpy as jnp
from jax import lax
from jax.experimental import pallas as pl
from jax.experimental.pallas import tpu as pltpu
from jax.experimental.pallas import tpu_sc as plsc

B, N, D, L = 16, 128, 256, 512
_NC, _NS = 2, 16
_NW = _NC * _NS
_ROWS = B * N
_RPW = _ROWS // _NW
_Q = L // 128


def _hist_body(graph_hbm, counts_hbm, g_v, c_v, sem):
    wid = lax.axis_index("s") * _NC + lax.axis_index("c")
    base = wid * _RPW
    cp = pltpu.async_copy(graph_hbm.at[pl.ds(base, _RPW)], g_v, sem)
    zeros = jnp.zeros((16,), jnp.int32)

    @plsc.parallel_loop(0, _RPW, step=2, unroll=4)
    def zero_chunk(k):
        for h in range(2):
            for t in range(8):
                c_v[k + h, pl.ds(t * 16, 16)] = zeros

    cp.wait()
    one = jnp.full((16,), 1, jnp.int32)

    @plsc.parallel_loop(0, _RPW // 2, step=1, unroll=2)
    def scat_row(lr):
        lr_v = jnp.full((16,), lr, jnp.int32)
        for k in range(N // 16):
            labels = g_v[lr, pl.ds(k * 16, 16)]
            byte_shift = lax.shift_left(
                lax.shift_right_logical(labels, 7), 3)
            val = lax.shift_left(one, byte_shift)
            col_idx = lax.bitwise_and(labels, 127)
            plsc.addupdate_scatter(c_v, [lr_v, col_idx], val)

    half = _RPW // 2
    cp_lo = pltpu.async_copy(c_v.at[pl.ds(0, half)],
                             counts_hbm.at[pl.ds(base, half)], sem)

    @plsc.parallel_loop(half, _RPW, step=1, unroll=2)
    def scat_row_hi(lr):
        lr_v = jnp.full((16,), lr, jnp.int32)
        for k in range(N // 16):
            labels = g_v[lr, pl.ds(k * 16, 16)]
            byte_shift = lax.shift_left(
                lax.shift_right_logical(labels, 7), 3)
            val = lax.shift_left(one, byte_shift)
            col_idx = lax.bitwise_and(labels, 127)
            plsc.addupdate_scatter(c_v, [lr_v, col_idx], val)

    cp_hi = pltpu.async_copy(c_v.at[pl.ds(half, half)],
                             counts_hbm.at[pl.ds(base + half, half)], sem)
    cp_lo.wait()
    cp_hi.wait()


_hist = functools.partial(
    pl.kernel,
    mesh=plsc.VectorSubcoreMesh(core_axis_name="c", subcore_axis_name="s"),
    out_type=jax.ShapeDtypeStruct((_ROWS, 128), jnp.int32),
    scratch_types=[
        pltpu.VMEM((_RPW, N), jnp.int32),
        pltpu.VMEM((_RPW, 128), jnp.int32),
        pltpu.SemaphoreType.DMA,
    ],
    compiler_params=pltpu.CompilerParams(needs_layout_passes=False),
)(_hist_body)


def _tc_linear_body(f_ref, w0_ref, w1_ref, o_ref):
    f = f_ref[...]
    w = w0_ref[...] + w1_ref[...]
    o_ref[...] = (f + lax.dot_general(
        f, w, (((1,), (1,)), ((), ())), preferred_element_type=jnp.float32
    )).astype(jnp.bfloat16)


def _tc_bias_body(p_ref, bias_ref, c_ref, o_ref):
    o = p_ref[...].astype(jnp.float32)
    packed = c_ref[...]
    for q in range(_Q):
        cnt = lax.bitwise_and(
            lax.shift_right_logical(packed, 8 * q), 255
        ).astype(jnp.float32)
        o = o + jnp.dot(cnt, bias_ref[q], preferred_element_type=jnp.float32)
    o_ref[...] = o


def kernel(feature, graph, W0, W1, bias):
    g2 = graph.reshape(_ROWS, N).astype(jnp.int32)
    counts = _hist(g2)
    f2 = feature.reshape(_ROWS, D)
    bias4 = bias.reshape(_Q, 128, D)
    partial = pl.pallas_call(
        _tc_linear_body,
        out_shape=jax.ShapeDtypeStruct((_ROWS, D), jnp.bfloat16),
    )(f2, W0, W1)
    out = pl.pallas_call(
        _tc_bias_body,
        out_shape=jax.ShapeDtypeStruct((_ROWS, D), jnp.float32),
    )(partial, bias4, counts)
    return out.reshape(B, N, D)


# speedup vs baseline: 1.0131x; 1.0131x over previous
import functools

import jax
import jax.numpy as jnp
from jax import lax
from jax.experimental import pallas as pl
from jax.experimental.pallas import tpu as pltpu
from jax.experimental.pallas import tpu_sc as plsc

B, N, D, L = 16, 128, 256, 512
_NC, _NS = 2, 16
_NW = _NC * _NS
_ROWS = B * N
_RPW = _ROWS // _NW
_Q = L // 128


def _hist_body(graph_hbm, counts_hbm, g_v, c_v, sem):
    wid = lax.axis_index("s") * _NC + lax.axis_index("c")
    base = wid * _RPW
    cp = pltpu.async_copy(graph_hbm.at[pl.ds(base, _RPW)], g_v, sem)
    zeros = jnp.zeros((16,), jnp.int32)

    @plsc.parallel_loop(0, _RPW, step=2, unroll=4)
    def zero_chunk(k):
        for h in range(2):
            for t in range(8):
                c_v[k + h, pl.ds(t * 16, 16)] = zeros

    cp.wait()
    one = jnp.full((16,), 1, jnp.int32)

    @plsc.parallel_loop(0, _RPW, step=1, unroll=2)
    def scat_row(lr):
        lr_v = jnp.full((16,), lr, jnp.int32)
        for k in range(N // 16):
            labels = g_v[lr, pl.ds(k * 16, 16)]
            byte_shift = lax.shift_left(
                lax.shift_right_logical(labels, 7), 3)
            val = lax.shift_left(one, byte_shift)
            col_idx = lax.bitwise_and(labels, 127)
            plsc.addupdate_scatter(c_v, [lr_v, col_idx], val)

    pltpu.sync_copy(c_v, counts_hbm.at[pl.ds(base, _RPW)])


_hist = functools.partial(
    pl.kernel,
    mesh=plsc.VectorSubcoreMesh(core_axis_name="c", subcore_axis_name="s"),
    out_type=jax.ShapeDtypeStruct((_ROWS, 128), jnp.int32),
    scratch_types=[
        pltpu.VMEM((_RPW, N), jnp.int32),
        pltpu.VMEM((_RPW, 128), jnp.int32),
        pltpu.SemaphoreType.DMA,
    ],
    compiler_params=pltpu.CompilerParams(needs_layout_passes=False),
)(_hist_body)


def _tc_linear_body(f_ref, w0_ref, w1_ref, o_ref):
    f = f_ref[...]
    w = w0_ref[...] + w1_ref[...]
    o_ref[...] = (f + lax.dot_general(
        f, w, (((1,), (1,)), ((), ())), preferred_element_type=jnp.float32
    )).astype(jnp.bfloat16)


def _tc_bias_body(p_ref, bias_ref, c_ref, o_ref):
    o = p_ref[...].astype(jnp.float32)
    packed = c_ref[...]
    for q in range(_Q):
        cnt = lax.bitwise_and(
            lax.shift_right_logical(packed, 8 * q), 255
        ).astype(jnp.float32)
        o = o + jnp.dot(cnt, bias_ref[q], preferred_element_type=jnp.float32)
    o_ref[...] = o


def kernel(feature, graph, W0, W1, bias):
    g2 = graph.reshape(_ROWS, N).astype(jnp.int32)
    counts = _hist(g2)
    f2 = feature.reshape(_ROWS, D)
    bias4 = bias.reshape(_Q, 128, D)
    partial = pl.pallas_call(
        _tc_linear_body,
        out_shape=jax.ShapeDtypeStruct((_ROWS, D), jnp.bfloat16),
    )(f2, W0, W1)
    blk = _ROWS // 2
    out = pl.pallas_call(
        _tc_bias_body,
        grid=(2,),
        in_specs=[
            pl.BlockSpec((blk, D), lambda i: (i, 0)),
            pl.BlockSpec((_Q, 128, D), lambda i: (0, 0, 0)),
            pl.BlockSpec((blk, 128), lambda i: (i, 0)),
        ],
        out_specs=pl.BlockSpec((blk, D), lambda i: (i, 0)),
        out_shape=jax.ShapeDtypeStruct((_ROWS, D), jnp.float32),
    )(partial, bias4, counts)
    return out.reshape(B, N, D)
